# baseline (speedup 1.0000x reference)
import functools

import jax
import jax.numpy as jnp
from jax import lax
from jax.experimental import pallas as pl
from jax.experimental.pallas import tpu as pltpu
from jax.experimental.pallas import tpu_sc as plsc

_SIZES = (56, 56, 56, 56, 32)
_OFFS = (0, 56, 112, 168, 224)
_NBUF = 2
_BUF_ROWS = max(_SIZES)

_SP_CHUNK = 64
_SP_NBUF = 3


def _sc_body(
    B,
    S,
    D,
    rows_per_w,
    table_hbm,
    out_hbm,
    bufs,
    in_sems,
    out_sems,
    sp_buf,
    sp_in_sems,
    sp_out_sems,
):
    c = lax.axis_index("c")
    s = lax.axis_index("s")
    wid = s * 2 + c
    slab = lax.rem(wid, 8)
    quarter = lax.div(wid, 8)
    src_base = slab * 1024 + quarter * rows_per_w
    n_iter = len(_SIZES)
    n_stream_b = B - 1

    def in_copy(i):
        k = i % _NBUF
        return pltpu.make_async_copy(
            table_hbm.at[pl.ds(src_base + _OFFS[i], _SIZES[i]), :],
            bufs.at[k, pl.ds(0, _SIZES[i]), :],
            in_sems.at[k],
        )

    def out_copy(i, b):
        k = i % _NBUF
        return pltpu.make_async_copy(
            bufs.at[k, pl.ds(0, _SIZES[i]), :],
            out_hbm.at[pl.ds(b * S + src_base + _OFFS[i], _SIZES[i]), :],
            out_sems.at[k],
        )

    half = S // 2
    sp_iters = half // _SP_CHUNK
    sp_base = c * half

    def sp_in(j):
        k = j % _SP_NBUF
        return pltpu.make_async_copy(
            table_hbm.at[pl.ds(sp_base + j * _SP_CHUNK, _SP_CHUNK), :],
            sp_buf.at[k],
            sp_in_sems.at[k],
        )

    def sp_out(j):
        k = j % _SP_NBUF
        return pltpu.make_async_copy(
            sp_buf.at[k],
            out_hbm.at[pl.ds((B - 1) * S + sp_base + j * _SP_CHUNK, _SP_CHUNK), :],
            sp_out_sems.at[k],
        )

    @pl.when(s == 0)
    def _spmem_path():
        sp_in(0).start()
        for j in range(sp_iters):
            if j + 1 < sp_iters:
                if j + 1 >= _SP_NBUF:
                    sp_out(j + 1 - _SP_NBUF).wait()
                sp_in(j + 1).start()
            sp_in(j).wait()
            sp_out(j).start()
        for j in range(max(0, sp_iters - _SP_NBUF), sp_iters):
            sp_out(j).wait()

    in_copy(0).start()
    for i in range(n_iter):
        if i + 1 < n_iter:
            if i + 1 >= _NBUF:
                for b in range(n_stream_b):
                    out_copy(i + 1 - _NBUF, b).wait()
            in_copy(i + 1).start()
        in_copy(i).wait()
        for b in range(n_stream_b):
            out_copy(i, b).start()
    for i in range(max(0, n_iter - _NBUF), n_iter):
        for b in range(n_stream_b):
            out_copy(i, b).wait()


def kernel(x, pos_embeddings):
    B, S = x.shape
    D = pos_embeddings.shape[1]
    rows_per_w = S // 8 // 4
    mesh = plsc.VectorSubcoreMesh(core_axis_name="c", subcore_axis_name="s")
    k = pl.kernel(
        functools.partial(_sc_body, B, S, D, rows_per_w),
        out_type=jax.ShapeDtypeStruct((B * S, D), pos_embeddings.dtype),
        mesh=mesh,
        scratch_types=[
            pltpu.VMEM((_NBUF, _BUF_ROWS, D), pos_embeddings.dtype),
            pltpu.SemaphoreType.DMA((_NBUF,)),
            pltpu.SemaphoreType.DMA((_NBUF,)),
            pltpu.VMEM_SHARED((_SP_NBUF, _SP_CHUNK, D), pos_embeddings.dtype),
            pltpu.SemaphoreType.DMA((_SP_NBUF,)),
            pltpu.SemaphoreType.DMA((_SP_NBUF,)),
        ],
    )
    out_flat = k(pos_embeddings)
    return out_flat.reshape(B, S, D)


# speedup vs baseline: 1.3397x; 1.3397x over previous
import functools

import jax
import jax.numpy as jnp
from jax import lax
from jax.experimental import pallas as pl
from jax.experimental.pallas import tpu as pltpu
from jax.experimental.pallas import tpu_sc as plsc

_SIZES = (56, 56, 56, 56, 32)
_OFFS = (0, 56, 112, 168, 224)
_NBUF = 2
_BUF_ROWS = max(_SIZES)

_SP_CHUNK = 64
_SP_NBUF = 3
_SP_ROWS = 2048


def _sc_body(
    B,
    S,
    D,
    rows_per_w,
    table_hbm,
    out_hbm,
    bufs,
    in_sems,
    out_sems,
    sp_buf,
    sp_in_sems,
    sp_out_sems,
):
    c = lax.axis_index("c")
    s = lax.axis_index("s")
    wid = s * 2 + c
    slab = lax.rem(wid, 8)
    quarter = lax.div(wid, 8)
    src_base = slab * 1024 + quarter * rows_per_w
    n_iter = len(_SIZES)
    stream_owns_b3 = lax.rem(slab, 4) >= (_SP_ROWS // 1024)

    def in_copy(i):
        k = i % _NBUF
        return pltpu.make_async_copy(
            table_hbm.at[pl.ds(src_base + _OFFS[i], _SIZES[i]), :],
            bufs.at[k, pl.ds(0, _SIZES[i]), :],
            in_sems.at[k],
        )

    def out_copy(i, b):
        k = i % _NBUF
        return pltpu.make_async_copy(
            bufs.at[k, pl.ds(0, _SIZES[i]), :],
            out_hbm.at[pl.ds(b * S + src_base + _OFFS[i], _SIZES[i]), :],
            out_sems.at[k],
        )

    half = S // 2
    sp_iters = _SP_ROWS // _SP_CHUNK
    sp_base = c * half

    def sp_in(j):
        k = j % _SP_NBUF
        return pltpu.make_async_copy(
            table_hbm.at[pl.ds(sp_base + j * _SP_CHUNK, _SP_CHUNK), :],
            sp_buf.at[k],
            sp_in_sems.at[k],
        )

    def sp_out(j):
        k = j % _SP_NBUF
        return pltpu.make_async_copy(
            sp_buf.at[k],
            out_hbm.at[pl.ds((B - 1) * S + sp_base + j * _SP_CHUNK, _SP_CHUNK), :],
            sp_out_sems.at[k],
        )

    @pl.when(s == 0)
    def _spmem_path():
        sp_in(0).start()
        for j in range(sp_iters):
            if j + 1 < sp_iters:
                if j + 1 >= _SP_NBUF:
                    sp_out(j + 1 - _SP_NBUF).wait()
                sp_in(j + 1).start()
            sp_in(j).wait()
            sp_out(j).start()
        for j in range(max(0, sp_iters - _SP_NBUF), sp_iters):
            sp_out(j).wait()

    def start_outs(i):
        for b in range(B - 1):
            out_copy(i, b).start()

        @pl.when(stream_owns_b3)
        def _():
            out_copy(i, B - 1).start()

    def wait_outs(i):
        for b in range(B - 1):
            out_copy(i, b).wait()

        @pl.when(stream_owns_b3)
        def _():
            out_copy(i, B - 1).wait()

    in_copy(0).start()
    for i in range(n_iter):
        if i + 1 < n_iter:
            if i + 1 >= _NBUF:
                wait_outs(i + 1 - _NBUF)
            in_copy(i + 1).start()
        in_copy(i).wait()
        start_outs(i)
    for i in range(max(0, n_iter - _NBUF), n_iter):
        wait_outs(i)


def kernel(x, pos_embeddings):
    B, S = x.shape
    D = pos_embeddings.shape[1]
    rows_per_w = S // 8 // 4
    mesh = plsc.VectorSubcoreMesh(core_axis_name="c", subcore_axis_name="s")
    k = pl.kernel(
        functools.partial(_sc_body, B, S, D, rows_per_w),
        out_type=jax.ShapeDtypeStruct((B * S, D), pos_embeddings.dtype),
        mesh=mesh,
        scratch_types=[
            pltpu.VMEM((_NBUF, _BUF_ROWS, D), pos_embeddings.dtype),
            pltpu.SemaphoreType.DMA((_NBUF,)),
            pltpu.SemaphoreType.DMA((_NBUF,)),
            pltpu.VMEM_SHARED((_SP_NBUF, _SP_CHUNK, D), pos_embeddings.dtype),
            pltpu.SemaphoreType.DMA((_SP_NBUF,)),
            pltpu.SemaphoreType.DMA((_SP_NBUF,)),
        ],
    )
    out_flat = k(pos_embeddings)
    return out_flat.reshape(B, S, D)


# speedup vs baseline: 1.6981x; 1.2675x over previous
import functools

import jax
import jax.numpy as jnp
from jax import lax
from jax.experimental import pallas as pl
from jax.experimental.pallas import tpu as pltpu
from jax.experimental.pallas import tpu_sc as plsc

_SIZES = (56, 56, 56, 56, 32)
_OFFS = (0, 56, 112, 168, 224)
_NBUF = 2
_BUF_ROWS = max(_SIZES)


def _sc_body(B, S, D, rows_per_w, table_hbm, out_hbm, bufs, in_sems, out_sems):
    wid = lax.axis_index("s") * 2 + lax.axis_index("c")
    slab = lax.rem(wid, 8)
    quarter = lax.div(wid, 8)
    src_base = slab * 1024 + quarter * rows_per_w
    n_iter = len(_SIZES)

    def in_copy(i):
        k = i % _NBUF
        return pltpu.make_async_copy(
            table_hbm.at[pl.ds(src_base + _OFFS[i], _SIZES[i]), :],
            bufs.at[k, pl.ds(0, _SIZES[i]), :],
            in_sems.at[k],
        )

    def out_copy(i, b):
        k = i % _NBUF
        return pltpu.make_async_copy(
            bufs.at[k, pl.ds(0, _SIZES[i]), :],
            out_hbm.at[pl.ds(b * S + src_base + _OFFS[i], _SIZES[i]), :],
            out_sems.at[k],
        )

    in_copy(0).start()
    for i in range(n_iter):
        if i + 1 < n_iter:
            if i + 1 >= _NBUF:
                for b in range(B):
                    out_copy(i + 1 - _NBUF, b).wait()
            in_copy(i + 1).start()
        in_copy(i).wait()
        for b in range(B):
            out_copy(i, b).start()
    for i in range(max(0, n_iter - _NBUF), n_iter):
        for b in range(B):
            out_copy(i, b).wait()


def kernel(x, pos_embeddings):
    B, S = x.shape
    D = pos_embeddings.shape[1]
    rows_per_w = S // 8 // 4
    mesh = plsc.VectorSubcoreMesh(core_axis_name="c", subcore_axis_name="s")
    k = pl.kernel(
        functools.partial(_sc_body, B, S, D, rows_per_w),
        out_type=jax.ShapeDtypeStruct((B * S, D), pos_embeddings.dtype),
        mesh=mesh,
        scratch_types=[
            pltpu.VMEM((_NBUF, _BUF_ROWS, D), pos_embeddings.dtype),
            pltpu.SemaphoreType.DMA((_NBUF,)),
            pltpu.SemaphoreType.DMA((_NBUF,)),
        ],
    )
    out_flat = k(pos_embeddings)
    return out_flat.reshape(B, S, D)
